# baseline (speedup 1.0000x reference)
import functools
import math

import jax
import jax.numpy as jnp
from jax.experimental import pallas as pl
from jax.experimental.pallas import tpu as pltpu

NEG = -1e30


def _proj_body(x_ref, wq_ref, bq_ref, wk_ref, bk_ref, wv_ref, bv_ref,
               g1_ref, be1_ref, nq_ref, nk_ref, vf_ref):
    x = x_ref[...]
    f32 = jnp.float32
    dot = functools.partial(jax.lax.dot_general,
                            dimension_numbers=(((1,), (0,)), ((), ())),
                            preferred_element_type=f32)
    q = dot(x, wq_ref[...]) + bq_ref[...]
    k = dot(x, wk_ref[...]) + bk_ref[...]
    qn = jnp.sqrt(jnp.sum(q * q, axis=1, keepdims=True))
    kn = jnp.sqrt(jnp.sum(k * k, axis=1, keepdims=True))
    nq_ref[...] = q / jnp.maximum(qn, 1e-12)
    nk_ref[...] = k / jnp.maximum(kn, 1e-12)
    m = jnp.mean(x, axis=1, keepdims=True)
    xc = x - m
    v = jnp.mean(xc * xc, axis=1, keepdims=True)
    xln = xc * jax.lax.rsqrt(v + 1e-5) * g1_ref[...] + be1_ref[...]
    vf_ref[...] = dot(xln.astype(jnp.bfloat16),
                      wv_ref[...].astype(jnp.bfloat16)) + bv_ref[...]


def _attn_body(nq_ref, nk_ref, vf_ref, x_ref, wo_ref, bo_ref, rw_ref,
               h1_ref, *, nk_count, heads):
    nq = nq_ref[0]
    nk = nk_ref[0]
    vf = vf_ref[0]
    dqk = nq.shape[1]
    d = vf.shape[1]
    hq = dqk // heads
    hv = d // heads
    dotT = functools.partial(jax.lax.dot_general,
                             dimension_numbers=(((1,), (1,)), ((), ())),
                             preferred_element_type=jnp.float32)
    dot = functools.partial(jax.lax.dot_general,
                            dimension_numbers=(((1,), (0,)), ((), ())),
                            preferred_element_type=jnp.float32)
    sim = dotT(nq, nk)
    work = sim
    for _ in range(nk_count - 1):
        mx = jnp.max(work, axis=1, keepdims=True)
        work = jnp.where(work == mx, NEG, work)
    thresh = jnp.max(work, axis=1, keepdims=True)
    mask = sim >= thresh
    scale = jnp.float32(1.0 / math.sqrt(hq))
    bf = jnp.bfloat16
    nqb = nq.astype(bf)
    nkb = nk.astype(bf)
    vfb = vf.astype(bf)
    outs = []
    for h in range(heads):
        qh = nqb[:, h * hq:(h + 1) * hq]
        kh = nkb[:, h * hq:(h + 1) * hq]
        lh = dotT(qh, kh) * scale
        lh = jnp.where(mask, lh, NEG)
        mh = jnp.max(lh, axis=1, keepdims=True)
        e = jnp.exp(lh - mh)
        e = jnp.where(mask, e, 0.0)
        att = e / jnp.sum(e, axis=1, keepdims=True)
        outs.append(dot(att.astype(bf), vfb[:, h * hv:(h + 1) * hv]))
    sa = jnp.concatenate(outs, axis=1)
    sa = dot(sa.astype(bf), wo_ref[...].astype(bf)) + bo_ref[...]
    h1_ref[0] = x_ref[0] + sa * rw_ref[...]


def _ffn_body(h1_ref, g2_ref, be2_ref, wf1_ref, bf1_ref, wf2_ref, bf2_ref,
              rw_ref, out_ref):
    h1 = h1_ref[...]
    dot = functools.partial(jax.lax.dot_general,
                            dimension_numbers=(((1,), (0,)), ((), ())),
                            preferred_element_type=jnp.float32)
    m = jnp.mean(h1, axis=1, keepdims=True)
    hc = h1 - m
    v = jnp.mean(hc * hc, axis=1, keepdims=True)
    hln = hc * jax.lax.rsqrt(v + 1e-5) * g2_ref[...] + be2_ref[...]
    bf = jnp.bfloat16
    a = dot(hln.astype(bf), wf1_ref[...].astype(bf)) + bf1_ref[...]
    g = 0.5 * a * (1.0 + jax.lax.erf(a * jnp.float32(1.0 / math.sqrt(2.0))))
    ff = dot(g.astype(bf), wf2_ref[...].astype(bf)) + bf2_ref[...]
    out_ref[...] = h1 + ff * rw_ref[...]


def kernel(x, Wq, bq, Wk, bk, Wv, bv, Wo, bo, g1, be1, g2, be2, Wf1, bf1,
           Wf2, bf2, res_w):
    B, N, D = x.shape
    DQK = Wq.shape[1]
    DFF = Wf1.shape[1]
    H = 8
    NKN = 16
    BN = B * N
    f32 = jnp.float32

    x2 = x.reshape(BN, D)
    row = lambda a: a.reshape(1, -1)
    rw = res_w.reshape(1, 1)

    RA = 512
    nq2, nk2, vf2 = pl.pallas_call(
        _proj_body,
        grid=(BN // RA,),
        in_specs=[
            pl.BlockSpec((RA, D), lambda i: (i, 0)),
            pl.BlockSpec((D, DQK), lambda i: (0, 0)),
            pl.BlockSpec((1, DQK), lambda i: (0, 0)),
            pl.BlockSpec((D, DQK), lambda i: (0, 0)),
            pl.BlockSpec((1, DQK), lambda i: (0, 0)),
            pl.BlockSpec((D, D), lambda i: (0, 0)),
            pl.BlockSpec((1, D), lambda i: (0, 0)),
            pl.BlockSpec((1, D), lambda i: (0, 0)),
            pl.BlockSpec((1, D), lambda i: (0, 0)),
        ],
        out_specs=[
            pl.BlockSpec((RA, DQK), lambda i: (i, 0)),
            pl.BlockSpec((RA, DQK), lambda i: (i, 0)),
            pl.BlockSpec((RA, D), lambda i: (i, 0)),
        ],
        out_shape=[
            jax.ShapeDtypeStruct((BN, DQK), f32),
            jax.ShapeDtypeStruct((BN, DQK), f32),
            jax.ShapeDtypeStruct((BN, D), f32),
        ],
    )(x2, Wq, row(bq), Wk, row(bk), Wv, row(bv), row(g1), row(be1))

    nq3 = nq2.reshape(B, N, DQK)
    nk3 = nk2.reshape(B, N, DQK)
    vf3 = vf2.reshape(B, N, D)

    RC = 256
    h1 = pl.pallas_call(
        functools.partial(_attn_body, nk_count=NKN, heads=H),
        grid=(B, N // RC),
        in_specs=[
            pl.BlockSpec((1, RC, DQK), lambda b, i: (b, i, 0)),
            pl.BlockSpec((1, N, DQK), lambda b, i: (b, 0, 0)),
            pl.BlockSpec((1, N, D), lambda b, i: (b, 0, 0)),
            pl.BlockSpec((1, RC, D), lambda b, i: (b, i, 0)),
            pl.BlockSpec((D, D), lambda b, i: (0, 0)),
            pl.BlockSpec((1, D), lambda b, i: (0, 0)),
            pl.BlockSpec((1, 1), lambda b, i: (0, 0)),
        ],
        out_specs=pl.BlockSpec((1, RC, D), lambda b, i: (b, i, 0)),
        out_shape=jax.ShapeDtypeStruct((B, N, D), f32),
    )(nq3, nk3, vf3, x, Wo, row(bo), rw)

    h12 = h1.reshape(BN, D)
    RD = 512
    out = pl.pallas_call(
        _ffn_body,
        grid=(BN // RD,),
        in_specs=[
            pl.BlockSpec((RD, D), lambda i: (i, 0)),
            pl.BlockSpec((1, D), lambda i: (0, 0)),
            pl.BlockSpec((1, D), lambda i: (0, 0)),
            pl.BlockSpec((D, DFF), lambda i: (0, 0)),
            pl.BlockSpec((1, DFF), lambda i: (0, 0)),
            pl.BlockSpec((DFF, D), lambda i: (0, 0)),
            pl.BlockSpec((1, D), lambda i: (0, 0)),
            pl.BlockSpec((1, 1), lambda i: (0, 0)),
        ],
        out_specs=pl.BlockSpec((RD, D), lambda i: (i, 0)),
        out_shape=jax.ShapeDtypeStruct((BN, D), f32),
    )(h12, row(g2), row(be2), Wf1, row(bf1), Wf2, row(bf2), rw)

    return out.reshape(B, N, D)


# speedup vs baseline: 1.4016x; 1.4016x over previous
import functools
import math

import jax
import jax.numpy as jnp
from jax.experimental import pallas as pl
from jax.experimental.pallas import tpu as pltpu

NEG = -1e30


def _proj_body(x_ref, wq_ref, bq_ref, wk_ref, bk_ref, wv_ref, bv_ref,
               g1_ref, be1_ref, nq_ref, nk_ref, vf_ref):
    x = x_ref[...]
    f32 = jnp.float32
    dot = functools.partial(jax.lax.dot_general,
                            dimension_numbers=(((1,), (0,)), ((), ())),
                            preferred_element_type=f32)
    q = dot(x, wq_ref[...]) + bq_ref[...]
    k = dot(x, wk_ref[...]) + bk_ref[...]
    qn = jnp.sqrt(jnp.sum(q * q, axis=1, keepdims=True))
    kn = jnp.sqrt(jnp.sum(k * k, axis=1, keepdims=True))
    nq_ref[...] = q / jnp.maximum(qn, 1e-12)
    nk_ref[...] = k / jnp.maximum(kn, 1e-12)
    m = jnp.mean(x, axis=1, keepdims=True)
    xc = x - m
    v = jnp.mean(xc * xc, axis=1, keepdims=True)
    xln = xc * jax.lax.rsqrt(v + 1e-5) * g1_ref[...] + be1_ref[...]
    vf_ref[...] = dot(xln.astype(jnp.bfloat16),
                      wv_ref[...].astype(jnp.bfloat16)) + bv_ref[...]


def _attn_body(nq_ref, nk_ref, vf_ref, x_ref, wo_ref, bo_ref, rw_ref,
               h1_ref, *, nk_count, heads):
    nq = nq_ref[0]
    nk = nk_ref[0]
    vf = vf_ref[0]
    dqk = nq.shape[1]
    d = vf.shape[1]
    hq = dqk // heads
    hv = d // heads
    dotT = functools.partial(jax.lax.dot_general,
                             dimension_numbers=(((1,), (1,)), ((), ())),
                             preferred_element_type=jnp.float32)
    dot = functools.partial(jax.lax.dot_general,
                            dimension_numbers=(((1,), (0,)), ((), ())),
                            preferred_element_type=jnp.float32)
    sim = dotT(nq, nk)
    work = sim
    for _ in range(nk_count - 1):
        mx = jnp.max(work, axis=1, keepdims=True)
        work = jnp.where(work == mx, NEG, work)
    thresh = jnp.max(work, axis=1, keepdims=True)
    nbias = jnp.where(sim >= thresh, 0.0, NEG)
    scale = jnp.float32(1.0 / math.sqrt(hq))
    bf = jnp.bfloat16
    nqb = (nq * scale).astype(bf)
    nkb = nk.astype(bf)
    outs = []
    for h in range(heads):
        qh = nqb[:, h * hq:(h + 1) * hq]
        kh = nkb[:, h * hq:(h + 1) * hq]
        e = jnp.exp(dotT(qh, kh) + nbias)
        s = jnp.sum(e, axis=1, keepdims=True)
        outs.append(dot(e, vf[:, h * hv:(h + 1) * hv]) / s)
    sa = jnp.concatenate(outs, axis=1)
    sa = dot(sa.astype(bf), wo_ref[...].astype(bf)) + bo_ref[...]
    h1_ref[0] = x_ref[0] + sa * rw_ref[...]


def _ffn_body(h1_ref, g2_ref, be2_ref, wf1_ref, bf1_ref, wf2_ref, bf2_ref,
              rw_ref, out_ref):
    h1 = h1_ref[...]
    dot = functools.partial(jax.lax.dot_general,
                            dimension_numbers=(((1,), (0,)), ((), ())),
                            preferred_element_type=jnp.float32)
    m = jnp.mean(h1, axis=1, keepdims=True)
    hc = h1 - m
    v = jnp.mean(hc * hc, axis=1, keepdims=True)
    hln = hc * jax.lax.rsqrt(v + 1e-5) * g2_ref[...] + be2_ref[...]
    bf = jnp.bfloat16
    a = dot(hln.astype(bf), wf1_ref[...].astype(bf)) + bf1_ref[...]
    g = 0.5 * a * (1.0 + jax.lax.erf(a * jnp.float32(1.0 / math.sqrt(2.0))))
    ff = dot(g.astype(bf), wf2_ref[...].astype(bf)) + bf2_ref[...]
    out_ref[...] = h1 + ff * rw_ref[...]


def kernel(x, Wq, bq, Wk, bk, Wv, bv, Wo, bo, g1, be1, g2, be2, Wf1, bf1,
           Wf2, bf2, res_w):
    B, N, D = x.shape
    DQK = Wq.shape[1]
    DFF = Wf1.shape[1]
    H = 8
    NKN = 16
    BN = B * N
    f32 = jnp.float32

    x2 = x.reshape(BN, D)
    row = lambda a: a.reshape(1, -1)
    rw = res_w.reshape(1, 1)

    RA = 512
    nq2, nk2, vf2 = pl.pallas_call(
        _proj_body,
        grid=(BN // RA,),
        in_specs=[
            pl.BlockSpec((RA, D), lambda i: (i, 0)),
            pl.BlockSpec((D, DQK), lambda i: (0, 0)),
            pl.BlockSpec((1, DQK), lambda i: (0, 0)),
            pl.BlockSpec((D, DQK), lambda i: (0, 0)),
            pl.BlockSpec((1, DQK), lambda i: (0, 0)),
            pl.BlockSpec((D, D), lambda i: (0, 0)),
            pl.BlockSpec((1, D), lambda i: (0, 0)),
            pl.BlockSpec((1, D), lambda i: (0, 0)),
            pl.BlockSpec((1, D), lambda i: (0, 0)),
        ],
        out_specs=[
            pl.BlockSpec((RA, DQK), lambda i: (i, 0)),
            pl.BlockSpec((RA, DQK), lambda i: (i, 0)),
            pl.BlockSpec((RA, D), lambda i: (i, 0)),
        ],
        out_shape=[
            jax.ShapeDtypeStruct((BN, DQK), f32),
            jax.ShapeDtypeStruct((BN, DQK), f32),
            jax.ShapeDtypeStruct((BN, D), f32),
        ],
    )(x2, Wq, row(bq), Wk, row(bk), Wv, row(bv), row(g1), row(be1))

    nq3 = nq2.reshape(B, N, DQK)
    nk3 = nk2.reshape(B, N, DQK)
    vf3 = vf2.reshape(B, N, D)

    RC = 256
    h1 = pl.pallas_call(
        functools.partial(_attn_body, nk_count=NKN, heads=H),
        grid=(B, N // RC),
        in_specs=[
            pl.BlockSpec((1, RC, DQK), lambda b, i: (b, i, 0)),
            pl.BlockSpec((1, N, DQK), lambda b, i: (b, 0, 0)),
            pl.BlockSpec((1, N, D), lambda b, i: (b, 0, 0)),
            pl.BlockSpec((1, RC, D), lambda b, i: (b, i, 0)),
            pl.BlockSpec((D, D), lambda b, i: (0, 0)),
            pl.BlockSpec((1, D), lambda b, i: (0, 0)),
            pl.BlockSpec((1, 1), lambda b, i: (0, 0)),
        ],
        out_specs=pl.BlockSpec((1, RC, D), lambda b, i: (b, i, 0)),
        out_shape=jax.ShapeDtypeStruct((B, N, D), f32),
    )(nq3, nk3, vf3, x, Wo, row(bo), rw)

    h12 = h1.reshape(BN, D)
    RD = 512
    out = pl.pallas_call(
        _ffn_body,
        grid=(BN // RD,),
        in_specs=[
            pl.BlockSpec((RD, D), lambda i: (i, 0)),
            pl.BlockSpec((1, D), lambda i: (0, 0)),
            pl.BlockSpec((1, D), lambda i: (0, 0)),
            pl.BlockSpec((D, DFF), lambda i: (0, 0)),
            pl.BlockSpec((1, DFF), lambda i: (0, 0)),
            pl.BlockSpec((DFF, D), lambda i: (0, 0)),
            pl.BlockSpec((1, D), lambda i: (0, 0)),
            pl.BlockSpec((1, 1), lambda i: (0, 0)),
        ],
        out_specs=pl.BlockSpec((RD, D), lambda i: (i, 0)),
        out_shape=jax.ShapeDtypeStruct((BN, D), f32),
    )(h12, row(g2), row(be2), Wf1, row(bf1), Wf2, row(bf2), rw)

    return out.reshape(B, N, D)


# speedup vs baseline: 1.5218x; 1.0857x over previous
import functools
import math

import jax
import jax.numpy as jnp
from jax.experimental import pallas as pl
from jax.experimental.pallas import tpu as pltpu

NEG = -1e30


def _proj_body(x_ref, wq_ref, bq_ref, wk_ref, bk_ref, wv_ref, bv_ref,
               g1_ref, be1_ref, nq_ref, nk_ref, vf_ref):
    x = x_ref[...]
    f32 = jnp.float32
    dot = functools.partial(jax.lax.dot_general,
                            dimension_numbers=(((1,), (0,)), ((), ())),
                            preferred_element_type=f32)
    q = dot(x, wq_ref[...]) + bq_ref[...]
    k = dot(x, wk_ref[...]) + bk_ref[...]
    qn = jnp.sqrt(jnp.sum(q * q, axis=1, keepdims=True))
    kn = jnp.sqrt(jnp.sum(k * k, axis=1, keepdims=True))
    nq_ref[...] = q / jnp.maximum(qn, 1e-12)
    nk_ref[...] = k / jnp.maximum(kn, 1e-12)
    m = jnp.mean(x, axis=1, keepdims=True)
    xc = x - m
    v = jnp.mean(xc * xc, axis=1, keepdims=True)
    xln = xc * jax.lax.rsqrt(v + 1e-5) * g1_ref[...] + be1_ref[...]
    vf_ref[...] = dot(xln.astype(jnp.bfloat16),
                      wv_ref[...].astype(jnp.bfloat16)) + bv_ref[...]


def _attn_body(nq_ref, nk_ref, vf_ref, x_ref, wo_ref, bo_ref, rw_ref,
               h1_ref, *, nk_count, heads):
    nq = nq_ref[0]
    nk = nk_ref[0]
    vf = vf_ref[0]
    dqk = nq.shape[1]
    d = vf.shape[1]
    hq = dqk // heads
    hv = d // heads
    dotT = functools.partial(jax.lax.dot_general,
                             dimension_numbers=(((1,), (1,)), ((), ())),
                             preferred_element_type=jnp.float32)
    dot = functools.partial(jax.lax.dot_general,
                            dimension_numbers=(((1,), (0,)), ((), ())),
                            preferred_element_type=jnp.float32)
    sim = dotT(nq, nk)
    work = sim
    for _ in range(nk_count - 1):
        mx = jnp.max(work, axis=1, keepdims=True)
        work = jnp.where(work == mx, NEG, work)
    thresh = jnp.max(work, axis=1, keepdims=True)
    nbias = jnp.where(sim >= thresh, 0.0, NEG)
    scale = jnp.float32(1.0 / math.sqrt(hq))
    bf = jnp.bfloat16
    nqb = (nq * scale).astype(bf)
    nkb = nk.astype(bf)
    ones = jnp.ones((vf.shape[0], 1), jnp.float32)
    outs = []
    for h in range(heads):
        qh = nqb[:, h * hq:(h + 1) * hq]
        kh = nkb[:, h * hq:(h + 1) * hq]
        e = jnp.exp(dotT(qh, kh) + nbias).astype(bf)
        vh = jnp.concatenate([vf[:, h * hv:(h + 1) * hv], ones], axis=1)
        os_ = dot(e, vh.astype(bf))
        outs.append(os_[:, :hv] / os_[:, hv:hv + 1])
    sa = jnp.concatenate(outs, axis=1)
    sa = dot(sa.astype(bf), wo_ref[...].astype(bf)) + bo_ref[...]
    h1_ref[0] = x_ref[0] + sa * rw_ref[...]


def _ffn_body(h1_ref, g2_ref, be2_ref, wf1_ref, bf1_ref, wf2_ref, bf2_ref,
              rw_ref, out_ref):
    h1 = h1_ref[...]
    dot = functools.partial(jax.lax.dot_general,
                            dimension_numbers=(((1,), (0,)), ((), ())),
                            preferred_element_type=jnp.float32)
    m = jnp.mean(h1, axis=1, keepdims=True)
    hc = h1 - m
    v = jnp.mean(hc * hc, axis=1, keepdims=True)
    hln = hc * jax.lax.rsqrt(v + 1e-5) * g2_ref[...] + be2_ref[...]
    bf = jnp.bfloat16
    a = dot(hln.astype(bf), wf1_ref[...].astype(bf)) + bf1_ref[...]
    g = 0.5 * a * (1.0 + jax.lax.erf(a * jnp.float32(1.0 / math.sqrt(2.0))))
    ff = dot(g.astype(bf), wf2_ref[...].astype(bf)) + bf2_ref[...]
    out_ref[...] = h1 + ff * rw_ref[...]


def kernel(x, Wq, bq, Wk, bk, Wv, bv, Wo, bo, g1, be1, g2, be2, Wf1, bf1,
           Wf2, bf2, res_w):
    B, N, D = x.shape
    DQK = Wq.shape[1]
    DFF = Wf1.shape[1]
    H = 8
    NKN = 16
    BN = B * N
    f32 = jnp.float32

    x2 = x.reshape(BN, D)
    row = lambda a: a.reshape(1, -1)
    rw = res_w.reshape(1, 1)

    RA = 512
    nq2, nk2, vf2 = pl.pallas_call(
        _proj_body,
        grid=(BN // RA,),
        in_specs=[
            pl.BlockSpec((RA, D), lambda i: (i, 0)),
            pl.BlockSpec((D, DQK), lambda i: (0, 0)),
            pl.BlockSpec((1, DQK), lambda i: (0, 0)),
            pl.BlockSpec((D, DQK), lambda i: (0, 0)),
            pl.BlockSpec((1, DQK), lambda i: (0, 0)),
            pl.BlockSpec((D, D), lambda i: (0, 0)),
            pl.BlockSpec((1, D), lambda i: (0, 0)),
            pl.BlockSpec((1, D), lambda i: (0, 0)),
            pl.BlockSpec((1, D), lambda i: (0, 0)),
        ],
        out_specs=[
            pl.BlockSpec((RA, DQK), lambda i: (i, 0)),
            pl.BlockSpec((RA, DQK), lambda i: (i, 0)),
            pl.BlockSpec((RA, D), lambda i: (i, 0)),
        ],
        out_shape=[
            jax.ShapeDtypeStruct((BN, DQK), f32),
            jax.ShapeDtypeStruct((BN, DQK), f32),
            jax.ShapeDtypeStruct((BN, D), f32),
        ],
    )(x2, Wq, row(bq), Wk, row(bk), Wv, row(bv), row(g1), row(be1))

    nq3 = nq2.reshape(B, N, DQK)
    nk3 = nk2.reshape(B, N, DQK)
    vf3 = vf2.reshape(B, N, D)

    RC = 512
    h1 = pl.pallas_call(
        functools.partial(_attn_body, nk_count=NKN, heads=H),
        grid=(B, N // RC),
        in_specs=[
            pl.BlockSpec((1, RC, DQK), lambda b, i: (b, i, 0)),
            pl.BlockSpec((1, N, DQK), lambda b, i: (b, 0, 0)),
            pl.BlockSpec((1, N, D), lambda b, i: (b, 0, 0)),
            pl.BlockSpec((1, RC, D), lambda b, i: (b, i, 0)),
            pl.BlockSpec((D, D), lambda b, i: (0, 0)),
            pl.BlockSpec((1, D), lambda b, i: (0, 0)),
            pl.BlockSpec((1, 1), lambda b, i: (0, 0)),
        ],
        out_specs=pl.BlockSpec((1, RC, D), lambda b, i: (b, i, 0)),
        out_shape=jax.ShapeDtypeStruct((B, N, D), f32),
    )(nq3, nk3, vf3, x, Wo, row(bo), rw)

    h12 = h1.reshape(BN, D)
    RD = 512
    out = pl.pallas_call(
        _ffn_body,
        grid=(BN // RD,),
        in_specs=[
            pl.BlockSpec((RD, D), lambda i: (i, 0)),
            pl.BlockSpec((1, D), lambda i: (0, 0)),
            pl.BlockSpec((1, D), lambda i: (0, 0)),
            pl.BlockSpec((D, DFF), lambda i: (0, 0)),
            pl.BlockSpec((1, DFF), lambda i: (0, 0)),
            pl.BlockSpec((DFF, D), lambda i: (0, 0)),
            pl.BlockSpec((1, D), lambda i: (0, 0)),
            pl.BlockSpec((1, 1), lambda i: (0, 0)),
        ],
        out_specs=pl.BlockSpec((RD, D), lambda i: (i, 0)),
        out_shape=jax.ShapeDtypeStruct((BN, D), f32),
    )(h12, row(g2), row(be2), Wf1, row(bf1), Wf2, row(bf2), rw)

    return out.reshape(B, N, D)


# speedup vs baseline: 1.5337x; 1.0078x over previous
import functools
import math

import jax
import jax.numpy as jnp
from jax.experimental import pallas as pl
from jax.experimental.pallas import tpu as pltpu

NEG = -1e30


def _proj_body(x_ref, wq_ref, bq_ref, wk_ref, bk_ref, wv_ref, bv_ref,
               g1_ref, be1_ref, nq_ref, nk_ref, vf_ref):
    x = x_ref[...]
    f32 = jnp.float32
    dot = functools.partial(jax.lax.dot_general,
                            dimension_numbers=(((1,), (0,)), ((), ())),
                            preferred_element_type=f32)
    q = dot(x, wq_ref[...]) + bq_ref[...]
    k = dot(x, wk_ref[...]) + bk_ref[...]
    qn = jnp.sqrt(jnp.sum(q * q, axis=1, keepdims=True))
    kn = jnp.sqrt(jnp.sum(k * k, axis=1, keepdims=True))
    nq_ref[...] = q / jnp.maximum(qn, 1e-12)
    nk_ref[...] = k / jnp.maximum(kn, 1e-12)
    m = jnp.mean(x, axis=1, keepdims=True)
    xc = x - m
    v = jnp.mean(xc * xc, axis=1, keepdims=True)
    xln = xc * jax.lax.rsqrt(v + 1e-5) * g1_ref[...] + be1_ref[...]
    vf_ref[...] = dot(xln.astype(jnp.bfloat16),
                      wv_ref[...].astype(jnp.bfloat16)) + bv_ref[...]


def _attn_body(nq_ref, nk_ref, vf_ref, x_ref, wo_ref, bo_ref, rw_ref,
               h1_ref, *, nk_count, heads):
    nq = nq_ref[0]
    nk = nk_ref[0]
    vf = vf_ref[0]
    dqk = nq.shape[1]
    d = vf.shape[1]
    hq = dqk // heads
    hv = d // heads
    dotT = functools.partial(jax.lax.dot_general,
                             dimension_numbers=(((1,), (1,)), ((), ())),
                             preferred_element_type=jnp.float32)
    dot = functools.partial(jax.lax.dot_general,
                            dimension_numbers=(((1,), (0,)), ((), ())),
                            preferred_element_type=jnp.float32)
    sim = dotT(nq, nk)
    work = sim
    for _ in range(nk_count - 1):
        mx = jnp.max(work, axis=1, keepdims=True)
        work = jnp.where(work == mx, NEG, work)
    thresh = jnp.max(work, axis=1, keepdims=True)
    bf = jnp.bfloat16
    maskb = jnp.where(sim >= thresh, 1.0, 0.0).astype(bf)
    scale = jnp.float32(1.0 / math.sqrt(hq))
    nqb = (nq * scale).astype(bf)
    nkb = nk.astype(bf)
    ones = jnp.ones((vf.shape[0], 1), jnp.float32)
    outs = []
    for h in range(heads):
        qh = nqb[:, h * hq:(h + 1) * hq]
        kh = nkb[:, h * hq:(h + 1) * hq]
        e = (jnp.exp(dotT(qh, kh)) * maskb).astype(bf)
        vh = jnp.concatenate([vf[:, h * hv:(h + 1) * hv], ones], axis=1)
        os_ = dot(e, vh.astype(bf))
        outs.append(os_[:, :hv] / os_[:, hv:hv + 1])
    sa = jnp.concatenate(outs, axis=1)
    sa = dot(sa.astype(bf), wo_ref[...].astype(bf)) + bo_ref[...]
    h1_ref[0] = x_ref[0] + sa * rw_ref[...]


def _ffn_body(h1_ref, g2_ref, be2_ref, wf1_ref, bf1_ref, wf2_ref, bf2_ref,
              rw_ref, out_ref):
    h1 = h1_ref[...]
    dot = functools.partial(jax.lax.dot_general,
                            dimension_numbers=(((1,), (0,)), ((), ())),
                            preferred_element_type=jnp.float32)
    m = jnp.mean(h1, axis=1, keepdims=True)
    hc = h1 - m
    v = jnp.mean(hc * hc, axis=1, keepdims=True)
    hln = hc * jax.lax.rsqrt(v + 1e-5) * g2_ref[...] + be2_ref[...]
    bf = jnp.bfloat16
    a = dot(hln.astype(bf), wf1_ref[...].astype(bf)) + bf1_ref[...]
    g = 0.5 * a * (1.0 + jax.lax.erf(a * jnp.float32(1.0 / math.sqrt(2.0))))
    ff = dot(g.astype(bf), wf2_ref[...].astype(bf)) + bf2_ref[...]
    out_ref[...] = h1 + ff * rw_ref[...]


def kernel(x, Wq, bq, Wk, bk, Wv, bv, Wo, bo, g1, be1, g2, be2, Wf1, bf1,
           Wf2, bf2, res_w):
    B, N, D = x.shape
    DQK = Wq.shape[1]
    DFF = Wf1.shape[1]
    H = 8
    NKN = 16
    BN = B * N
    f32 = jnp.float32

    x2 = x.reshape(BN, D)
    row = lambda a: a.reshape(1, -1)
    rw = res_w.reshape(1, 1)

    RA = 512
    nq2, nk2, vf2 = pl.pallas_call(
        _proj_body,
        grid=(BN // RA,),
        in_specs=[
            pl.BlockSpec((RA, D), lambda i: (i, 0)),
            pl.BlockSpec((D, DQK), lambda i: (0, 0)),
            pl.BlockSpec((1, DQK), lambda i: (0, 0)),
            pl.BlockSpec((D, DQK), lambda i: (0, 0)),
            pl.BlockSpec((1, DQK), lambda i: (0, 0)),
            pl.BlockSpec((D, D), lambda i: (0, 0)),
            pl.BlockSpec((1, D), lambda i: (0, 0)),
            pl.BlockSpec((1, D), lambda i: (0, 0)),
            pl.BlockSpec((1, D), lambda i: (0, 0)),
        ],
        out_specs=[
            pl.BlockSpec((RA, DQK), lambda i: (i, 0)),
            pl.BlockSpec((RA, DQK), lambda i: (i, 0)),
            pl.BlockSpec((RA, D), lambda i: (i, 0)),
        ],
        out_shape=[
            jax.ShapeDtypeStruct((BN, DQK), f32),
            jax.ShapeDtypeStruct((BN, DQK), f32),
            jax.ShapeDtypeStruct((BN, D), f32),
        ],
    )(x2, Wq, row(bq), Wk, row(bk), Wv, row(bv), row(g1), row(be1))

    nq3 = nq2.reshape(B, N, DQK)
    nk3 = nk2.reshape(B, N, DQK)
    vf3 = vf2.reshape(B, N, D)

    RC = 512
    h1 = pl.pallas_call(
        functools.partial(_attn_body, nk_count=NKN, heads=H),
        grid=(B, N // RC),
        in_specs=[
            pl.BlockSpec((1, RC, DQK), lambda b, i: (b, i, 0)),
            pl.BlockSpec((1, N, DQK), lambda b, i: (b, 0, 0)),
            pl.BlockSpec((1, N, D), lambda b, i: (b, 0, 0)),
            pl.BlockSpec((1, RC, D), lambda b, i: (b, i, 0)),
            pl.BlockSpec((D, D), lambda b, i: (0, 0)),
            pl.BlockSpec((1, D), lambda b, i: (0, 0)),
            pl.BlockSpec((1, 1), lambda b, i: (0, 0)),
        ],
        out_specs=pl.BlockSpec((1, RC, D), lambda b, i: (b, i, 0)),
        out_shape=jax.ShapeDtypeStruct((B, N, D), f32),
    )(nq3, nk3, vf3, x, Wo, row(bo), rw)

    h12 = h1.reshape(BN, D)
    RD = 512
    out = pl.pallas_call(
        _ffn_body,
        grid=(BN // RD,),
        in_specs=[
            pl.BlockSpec((RD, D), lambda i: (i, 0)),
            pl.BlockSpec((1, D), lambda i: (0, 0)),
            pl.BlockSpec((1, D), lambda i: (0, 0)),
            pl.BlockSpec((D, DFF), lambda i: (0, 0)),
            pl.BlockSpec((1, DFF), lambda i: (0, 0)),
            pl.BlockSpec((DFF, D), lambda i: (0, 0)),
            pl.BlockSpec((1, D), lambda i: (0, 0)),
            pl.BlockSpec((1, 1), lambda i: (0, 0)),
        ],
        out_specs=pl.BlockSpec((RD, D), lambda i: (i, 0)),
        out_shape=jax.ShapeDtypeStruct((BN, D), f32),
    )(h12, row(g2), row(be2), Wf1, row(bf1), Wf2, row(bf2), rw)

    return out.reshape(B, N, D)


# speedup vs baseline: 1.5750x; 1.0270x over previous
import functools
import math

import jax
import jax.numpy as jnp
from jax.experimental import pallas as pl
from jax.experimental.pallas import tpu as pltpu

NEG = -1e30


def _proj_body(x_ref, wq_ref, bq_ref, wk_ref, bk_ref, wv_ref, bv_ref,
               g1_ref, be1_ref, nq_ref, nk_ref, vf_ref):
    x = x_ref[...]
    f32 = jnp.float32
    dot = functools.partial(jax.lax.dot_general,
                            dimension_numbers=(((1,), (0,)), ((), ())),
                            preferred_element_type=f32)
    q = dot(x, wq_ref[...]) + bq_ref[...]
    k = dot(x, wk_ref[...]) + bk_ref[...]
    qn = jnp.sqrt(jnp.sum(q * q, axis=1, keepdims=True))
    kn = jnp.sqrt(jnp.sum(k * k, axis=1, keepdims=True))
    nq_ref[...] = q / jnp.maximum(qn, 1e-12)
    nk_ref[...] = k / jnp.maximum(kn, 1e-12)
    m = jnp.mean(x, axis=1, keepdims=True)
    xc = x - m
    v = jnp.mean(xc * xc, axis=1, keepdims=True)
    xln = xc * jax.lax.rsqrt(v + 1e-5) * g1_ref[...] + be1_ref[...]
    vf_ref[...] = dot(xln.astype(jnp.bfloat16),
                      wv_ref[...].astype(jnp.bfloat16)) + bv_ref[...]


def _attn_body(nq_ref, nk_ref, vf_ref, x_ref, wo_ref, bo_ref, rw_ref,
               g2_ref, be2_ref, wf1_ref, bf1_ref, wf2_ref, bf2_ref,
               out_ref, *, nk_count, heads):
    nq = nq_ref[0]
    nk = nk_ref[0]
    vf = vf_ref[0]
    dqk = nq.shape[1]
    d = vf.shape[1]
    hq = dqk // heads
    hv = d // heads
    dotT = functools.partial(jax.lax.dot_general,
                             dimension_numbers=(((1,), (1,)), ((), ())),
                             preferred_element_type=jnp.float32)
    dot = functools.partial(jax.lax.dot_general,
                            dimension_numbers=(((1,), (0,)), ((), ())),
                            preferred_element_type=jnp.float32)
    sim = dotT(nq, nk)
    work = sim
    for _ in range(nk_count - 1):
        mx = jnp.max(work, axis=1, keepdims=True)
        work = jnp.where(work == mx, NEG, work)
    thresh = jnp.max(work, axis=1, keepdims=True)
    bf = jnp.bfloat16
    maskb = jnp.where(sim >= thresh, 1.0, 0.0).astype(bf)
    scale = jnp.float32(1.0 / math.sqrt(hq))
    nqb = (nq * scale).astype(bf)
    nkb = nk.astype(bf)
    ones = jnp.ones((vf.shape[0], 1), jnp.float32)
    outs = []
    for h in range(heads):
        qh = nqb[:, h * hq:(h + 1) * hq]
        kh = nkb[:, h * hq:(h + 1) * hq]
        e = (jnp.exp(dotT(qh, kh)) * maskb).astype(bf)
        vh = jnp.concatenate([vf[:, h * hv:(h + 1) * hv], ones], axis=1)
        os_ = dot(e, vh.astype(bf))
        outs.append(os_[:, :hv] / os_[:, hv:hv + 1])
    sa = jnp.concatenate(outs, axis=1)
    sa = dot(sa.astype(bf), wo_ref[...].astype(bf)) + bo_ref[...]
    h1 = x_ref[0] + sa * rw_ref[...]
    m = jnp.mean(h1, axis=1, keepdims=True)
    hc = h1 - m
    v = jnp.mean(hc * hc, axis=1, keepdims=True)
    hln = hc * jax.lax.rsqrt(v + 1e-5) * g2_ref[...] + be2_ref[...]
    a = dot(hln.astype(bf), wf1_ref[...].astype(bf)) + bf1_ref[...]
    g = 0.5 * a * (1.0 + jax.lax.erf(a * jnp.float32(1.0 / math.sqrt(2.0))))
    ff = dot(g.astype(bf), wf2_ref[...].astype(bf)) + bf2_ref[...]
    out_ref[0] = h1 + ff * rw_ref[...]


def kernel(x, Wq, bq, Wk, bk, Wv, bv, Wo, bo, g1, be1, g2, be2, Wf1, bf1,
           Wf2, bf2, res_w):
    B, N, D = x.shape
    DQK = Wq.shape[1]
    DFF = Wf1.shape[1]
    H = 8
    NKN = 16
    BN = B * N
    f32 = jnp.float32

    x2 = x.reshape(BN, D)
    row = lambda a: a.reshape(1, -1)
    rw = res_w.reshape(1, 1)

    RA = 512
    nq2, nk2, vf2 = pl.pallas_call(
        _proj_body,
        grid=(BN // RA,),
        in_specs=[
            pl.BlockSpec((RA, D), lambda i: (i, 0)),
            pl.BlockSpec((D, DQK), lambda i: (0, 0)),
            pl.BlockSpec((1, DQK), lambda i: (0, 0)),
            pl.BlockSpec((D, DQK), lambda i: (0, 0)),
            pl.BlockSpec((1, DQK), lambda i: (0, 0)),
            pl.BlockSpec((D, D), lambda i: (0, 0)),
            pl.BlockSpec((1, D), lambda i: (0, 0)),
            pl.BlockSpec((1, D), lambda i: (0, 0)),
            pl.BlockSpec((1, D), lambda i: (0, 0)),
        ],
        out_specs=[
            pl.BlockSpec((RA, DQK), lambda i: (i, 0)),
            pl.BlockSpec((RA, DQK), lambda i: (i, 0)),
            pl.BlockSpec((RA, D), lambda i: (i, 0)),
        ],
        out_shape=[
            jax.ShapeDtypeStruct((BN, DQK), f32),
            jax.ShapeDtypeStruct((BN, DQK), f32),
            jax.ShapeDtypeStruct((BN, D), f32),
        ],
    )(x2, Wq, row(bq), Wk, row(bk), Wv, row(bv), row(g1), row(be1))

    nq3 = nq2.reshape(B, N, DQK)
    nk3 = nk2.reshape(B, N, DQK)
    vf3 = vf2.reshape(B, N, D)

    RC = 512
    out = pl.pallas_call(
        functools.partial(_attn_body, nk_count=NKN, heads=H),
        grid=(B, N // RC),
        in_specs=[
            pl.BlockSpec((1, RC, DQK), lambda b, i: (b, i, 0)),
            pl.BlockSpec((1, N, DQK), lambda b, i: (b, 0, 0)),
            pl.BlockSpec((1, N, D), lambda b, i: (b, 0, 0)),
            pl.BlockSpec((1, RC, D), lambda b, i: (b, i, 0)),
            pl.BlockSpec((D, D), lambda b, i: (0, 0)),
            pl.BlockSpec((1, D), lambda b, i: (0, 0)),
            pl.BlockSpec((1, 1), lambda b, i: (0, 0)),
            pl.BlockSpec((1, D), lambda b, i: (0, 0)),
            pl.BlockSpec((1, D), lambda b, i: (0, 0)),
            pl.BlockSpec((D, DFF), lambda b, i: (0, 0)),
            pl.BlockSpec((1, DFF), lambda b, i: (0, 0)),
            pl.BlockSpec((DFF, D), lambda b, i: (0, 0)),
            pl.BlockSpec((1, D), lambda b, i: (0, 0)),
        ],
        out_specs=pl.BlockSpec((1, RC, D), lambda b, i: (b, i, 0)),
        out_shape=jax.ShapeDtypeStruct((B, N, D), f32),
    )(nq3, nk3, vf3, x, Wo, row(bo), rw, row(g2), row(be2), Wf1, row(bf1),
      Wf2, row(bf2))

    return out
